# baseline (speedup 1.0000x reference)
import jax
import jax.numpy as jnp
from jax.experimental import pallas as pl
from jax.experimental.pallas import tpu as pltpu

_N = 20000
_C = 80
_K = 100
_THR = 0.01
_S = 125
_B = 160
_NEG = float("-inf")
_BIG = 2 ** 30


def _filter_kernel(scores_ref, feat_ref, featout_ref, scoreout_ref,
                   labelout_ref, work_ref):
    lane_row = jax.lax.broadcasted_iota(jnp.int32, (1, 128), 1)
    a_col = jax.lax.broadcasted_iota(jnp.int32, (128, 1), 0)
    r_seg = jax.lax.broadcasted_iota(jnp.int32, (_B, _C), 0)
    l_seg = jax.lax.broadcasted_iota(jnp.int32, (_B, _C), 1)

    count = jnp.int32(0)
    segv = jnp.full((1, 128), _NEG, jnp.float32)
    segid = jnp.full((1, 128), _BIG, jnp.int32)
    for seg in range(_S):
        s = scores_ref[seg * _B:(seg + 1) * _B, :]
        count = count + jnp.sum((s > _THR).astype(jnp.int32))
        work_ref[seg * _B:(seg + 1) * _B, :] = s
        mx = jnp.max(s)
        fi = l_seg * _N + (seg * _B + r_seg)
        sid = jnp.min(jnp.where(s == mx, fi, _BIG))
        segv = jnp.where(lane_row == seg, mx, segv)
        segid = jnp.where(lane_row == seg, sid, segid)

    def body(j, carry):
        id_row, id_col, sc_row, sc_col, segv, segid = carry

        m = jnp.max(segv)
        tied = segv == m
        idx = jnp.min(jnp.where(tied, segid, _BIG))
        seg = jnp.min(jnp.where(tied & (segid == idx), lane_row, _BIG))

        base = seg * _B
        w = work_ref[pl.ds(base, _B), :]
        fi = l_seg * _N + (base + r_seg)
        w = jnp.where(fi == idx, _NEG, w)
        work_ref[pl.ds(base, _B), :] = w
        mx = jnp.max(w)
        sid = jnp.min(jnp.where(w == mx, fi, _BIG))
        segv = jnp.where(lane_row == seg, mx, segv)
        segid = jnp.where(lane_row == seg, sid, segid)

        id_row = jnp.where(lane_row == j, idx, id_row)
        id_col = jnp.where(a_col == j, idx, id_col)
        sc_row = jnp.where(lane_row == j, m, sc_row)
        sc_col = jnp.where(a_col == j, m, sc_col)
        return id_row, id_col, sc_row, sc_col, segv, segid

    init = (jnp.zeros((1, 128), jnp.int32), jnp.zeros((128, 1), jnp.int32),
            jnp.full((1, 128), _NEG, jnp.float32),
            jnp.full((128, 1), _NEG, jnp.float32), segv, segid)
    id_row, id_col, sc_row, sc_col, segv, segid = jax.lax.fori_loop(
        0, _K, body, init)

    valid_row = lane_row < count
    valid_col = a_col < count
    rank_row = jnp.sum((valid_col & (id_col < id_row)).astype(jnp.int32),
                       axis=0, keepdims=True)
    place = (rank_row == a_col) & valid_row
    ordered_id_col = jnp.sum(jnp.where(place, id_row, 0), axis=1,
                             keepdims=True)
    ordered_sc_col = jnp.sum(jnp.where(place, sc_row, 0.0), axis=1,
                             keepdims=True)

    use_topk = count > _K
    fid_col = jnp.where(use_topk, id_col, ordered_id_col)
    fsc_col = jnp.where(use_topk, sc_col, ordered_sc_col)
    valid_out = a_col < jnp.minimum(count, _K)

    box_idx = fid_col % _N
    label = fid_col // _N
    i_row = jax.lax.broadcasted_iota(jnp.int32, (128, _N), 1)
    onehot = (box_idx == i_row).astype(jnp.float32)
    gat = jax.lax.dot_general(
        onehot, feat_ref[...], (((1,), (0,)), ((), ())),
        preferred_element_type=jnp.float32,
        precision=jax.lax.Precision.HIGHEST)
    featout_ref[...] = jnp.where(valid_out, gat, jnp.float32(-1.0))
    scoreout_ref[...] = jnp.where(valid_out, fsc_col, jnp.float32(-1.0))
    labelout_ref[...] = jnp.where(valid_out, label, jnp.int32(-1))


@jax.jit
def kernel(boxes, classification, rotation, translation):
    feat = jnp.concatenate([boxes, rotation, translation], axis=1)

    featout, scores, labels = pl.pallas_call(
        _filter_kernel,
        out_shape=[jax.ShapeDtypeStruct((128, 10), jnp.float32),
                   jax.ShapeDtypeStruct((128, 1), jnp.float32),
                   jax.ShapeDtypeStruct((128, 1), jnp.int32)],
        scratch_shapes=[pltpu.VMEM((_N, _C), jnp.float32)],
        compiler_params=pltpu.CompilerParams(
            vmem_limit_bytes=100 * 1024 * 1024),
    )(classification, feat)

    boxes_out = featout[:_K, :4]
    rotation_out = featout[:_K, 4:7]
    translation_out = featout[:_K, 7:10]
    scores_out = scores[:_K, 0]
    labels_out = labels[:_K, 0]
    return boxes_out, scores_out, labels_out, rotation_out, translation_out
